# Initial kernel scaffold; baseline (speedup 1.0000x reference)
import jax
import jax.numpy as jnp
from jax.experimental import pallas as pl


def kernel(images):
    raise NotImplementedError("write your pallas kernel here")


# speedup vs baseline: 11.8221x; 11.8221x over previous
import functools

import jax
import jax.numpy as jnp
from jax import lax
from jax.experimental import pallas as pl
from jax.experimental.pallas import tpu as pltpu
from jax.experimental.pallas import tpu_sc as plsc

K = 3
B, H, W, C = 2, 224, 224, 96
WCHUNK = 32
NCHUNKS = W // WCHUNK
TILES_TOTAL = B * H * NCHUNKS
NWORKERS = 32
PER_WORKER = TILES_TOTAL // NWORKERS


def _body(padded_hbm, out_hbm, in_local):
    cid = lax.axis_index("c")
    sid = lax.axis_index("s")
    wid = sid * 2 + cid

    def step(i, carry):
        t = i * NWORKERS + wid
        b = t // (H * NCHUNKS)
        rem = t - b * (H * NCHUNKS)
        r = rem // NCHUNKS
        cc = rem - r * NCHUNKS
        c0 = cc * WCHUNK

        pltpu.sync_copy(
            padded_hbm.at[b, pl.ds(r, K), pl.ds(c0, WCHUNK + K - 1), :],
            in_local,
        )
        for u in range(K):
            for v in range(K):
                pltpu.sync_copy(
                    in_local.at[u, pl.ds(v, WCHUNK), :],
                    out_hbm.at[b, r, pl.ds(c0, WCHUNK),
                               pl.ds((u * K + v) * C, C)],
                )
        return carry

    lax.fori_loop(0, PER_WORKER, step, 0)


@jax.jit
def kernel(images):
    p = (K - 1) // 2
    padded = jnp.pad(images, ((0, 0), (p, p), (p, p), (0, 0)))
    run = pl.kernel(
        _body,
        out_type=jax.ShapeDtypeStruct((B, H, W, K * K * C), jnp.float32),
        mesh=plsc.VectorSubcoreMesh(core_axis_name="c", subcore_axis_name="s"),
        scratch_types=[
            pltpu.VMEM((K, WCHUNK + K - 1, C), jnp.float32),
        ],
        compiler_params=pltpu.CompilerParams(use_tc_tiling_on_sc=False),
    )
    return run(padded)


# speedup vs baseline: 13.3028x; 1.1252x over previous
import jax
import jax.numpy as jnp
from jax import lax
from jax.experimental import pallas as pl
from jax.experimental.pallas import tpu as pltpu
from jax.experimental.pallas import tpu_sc as plsc

K = 3
B, H, W, C = 2, 224, 224, 96
WCHUNK = 112
NCHUNKS = W // WCHUNK
TILES_TOTAL = B * H * NCHUNKS
NWORKERS = 32
PER_WORKER = TILES_TOTAL // NWORKERS
HALO = WCHUNK + K - 1


def _decode(t):
    b = t // (H * NCHUNKS)
    rem = t - b * (H * NCHUNKS)
    r = rem // NCHUNKS
    cc = rem - r * NCHUNKS
    return b, r, cc * WCHUNK


def _body(padded_hbm, out_hbm, buf0, buf1, gsem0, gsem1, ssem0, ssem1):
    cid = lax.axis_index("c")
    sid = lax.axis_index("s")
    wid = sid * 2 + cid

    def gather(t, buf, sem):
        b, r, c0 = _decode(t)
        return pltpu.async_copy(
            padded_hbm.at[b, pl.ds(r, K), pl.ds(c0, HALO), :], buf, sem)

    def stores(t, buf, sem):
        b, r, c0 = _decode(t)
        return [
            pltpu.async_copy(
                buf.at[u, pl.ds(v, WCHUNK), :],
                out_hbm.at[b, r, pl.ds(c0, WCHUNK), pl.ds((u * K + v) * C, C)],
                sem)
            for u in range(K) for v in range(K)
        ]

    def step(j, carry):
        ta = (2 * j) * NWORKERS + wid
        tb = (2 * j + 1) * NWORKERS + wid
        ha = gather(ta, buf0, gsem0)
        hb = gather(tb, buf1, gsem1)
        ha.wait()
        hs_a = stores(ta, buf0, ssem0)
        hb.wait()
        hs_b = stores(tb, buf1, ssem1)
        for h in hs_a:
            h.wait()
        for h in hs_b:
            h.wait()
        return carry

    lax.fori_loop(0, PER_WORKER // 2, step, 0)


@jax.jit
def kernel(images):
    p = (K - 1) // 2
    padded = jnp.pad(images, ((0, 0), (p, p), (p, p), (0, 0)))
    run = pl.kernel(
        _body,
        out_type=jax.ShapeDtypeStruct((B, H, W, K * K * C), jnp.float32),
        mesh=plsc.VectorSubcoreMesh(core_axis_name="c", subcore_axis_name="s"),
        scratch_types=[
            pltpu.VMEM((K, HALO, C), jnp.float32),
            pltpu.VMEM((K, HALO, C), jnp.float32),
            pltpu.SemaphoreType.DMA,
            pltpu.SemaphoreType.DMA,
            pltpu.SemaphoreType.DMA,
            pltpu.SemaphoreType.DMA,
        ],
        compiler_params=pltpu.CompilerParams(use_tc_tiling_on_sc=False),
    )
    return run(padded)


# speedup vs baseline: 13.7051x; 1.0302x over previous
import jax
import jax.numpy as jnp
from jax import lax
from jax.experimental import pallas as pl
from jax.experimental.pallas import tpu as pltpu
from jax.experimental.pallas import tpu_sc as plsc

K = 3
B, H, W, C = 2, 224, 224, 96
WCHUNK = 112
NCHUNKS = W // WCHUNK
TILES_TOTAL = B * H * NCHUNKS
NWORKERS = 32
PER_WORKER = TILES_TOTAL // NWORKERS
HALO = WCHUNK + K - 1
VALID = WCHUNK + 1


def _decode(t):
    b = t // (H * NCHUNKS)
    rem = t - b * (H * NCHUNKS)
    r = rem // NCHUNKS
    cc = rem - r * NCHUNKS
    return b, r, cc


def _body(images_hbm, zrow_hbm, out_hbm, buf0, buf1, gsem0, gsem1, ssem0,
          ssem1):
    cid = lax.axis_index("c")
    sid = lax.axis_index("s")
    wid = sid * 2 + cid

    zv = jnp.zeros((16,), jnp.float32)
    for bf in (buf0, buf1):
        for u in range(K):
            for col in (0, HALO - 1):
                for kk in range(C // 16):
                    bf[u, col, pl.ds(16 * kk, 16)] = zv

    def gather(t, buf, sem):
        b, r, cc = _decode(t)
        c0 = cc * WCHUNK
        s_in = c0 - cc
        d0 = 1 - cc
        handles = []
        for u in range(K):
            dst = buf.at[u, pl.ds(d0, VALID), :]
            if u == 1:
                handles.append(
                    pltpu.async_copy(
                        images_hbm.at[b, r, pl.ds(s_in, VALID), :], dst, sem))
            else:
                row = r - 1 + u
                ok = (row >= 0) if u == 0 else (row < H)
                hs = []

                @pl.when(ok)
                def _(row=row, dst=dst, hs=hs):
                    hs.append(
                        pltpu.async_copy(
                            images_hbm.at[b, row, pl.ds(s_in, VALID), :],
                            dst, sem))

                @pl.when(jnp.logical_not(ok))
                def _(dst=dst):
                    pltpu.async_copy(zrow_hbm, dst, sem)

                handles.append(hs[0])
        return handles

    def stores(t, buf, sem):
        b, r, cc = _decode(t)
        c0 = cc * WCHUNK
        return [
            pltpu.async_copy(
                buf.at[u, pl.ds(v, WCHUNK), :],
                out_hbm.at[b, r, pl.ds(c0, WCHUNK), pl.ds((u * K + v) * C, C)],
                sem)
            for u in range(K) for v in range(K)
        ]

    def step(j, carry):
        ta = (2 * j) * NWORKERS + wid
        tb = (2 * j + 1) * NWORKERS + wid
        ha = gather(ta, buf0, gsem0)
        hb = gather(tb, buf1, gsem1)
        for h in ha:
            h.wait()
        hs_a = stores(ta, buf0, ssem0)
        for h in hb:
            h.wait()
        hs_b = stores(tb, buf1, ssem1)
        for h in hs_a:
            h.wait()
        for h in hs_b:
            h.wait()
        return carry

    lax.fori_loop(0, PER_WORKER // 2, step, 0)


@jax.jit
def kernel(images):
    zrow = jnp.zeros((VALID, C), jnp.float32)
    run = pl.kernel(
        _body,
        out_type=jax.ShapeDtypeStruct((B, H, W, K * K * C), jnp.float32),
        mesh=plsc.VectorSubcoreMesh(core_axis_name="c", subcore_axis_name="s"),
        scratch_types=[
            pltpu.VMEM((K, HALO, C), jnp.float32),
            pltpu.VMEM((K, HALO, C), jnp.float32),
            pltpu.SemaphoreType.DMA,
            pltpu.SemaphoreType.DMA,
            pltpu.SemaphoreType.DMA,
            pltpu.SemaphoreType.DMA,
        ],
        compiler_params=pltpu.CompilerParams(use_tc_tiling_on_sc=False),
    )
    return run(images, zrow)
